# baseline (speedup 1.0000x reference)
import functools
import math

import jax
import jax.numpy as jnp
from jax import lax
from jax.experimental import pallas as pl
from jax.experimental.pallas import tpu as pltpu
from jax.experimental.pallas import tpu_sc as plsc


def _make_segsum(n_rows, n_acc, e, w):
    info = plsc.get_sparse_core_info()
    nc, ns = info.num_cores, info.num_subcores
    nw = nc * ns
    chunk = 128
    per_w = e // nw
    n_chunks = per_w // chunk
    assert per_w * nw == e and n_chunks * chunk == per_w, (e, nw, chunk)
    rows_per_tile = n_acc // ns
    assert rows_per_tile * ns == n_acc and rows_per_tile % chunk == 0
    mesh = plsc.VectorSubcoreMesh(core_axis_name="c", subcore_axis_name="s")

    @functools.partial(
        pl.kernel, mesh=mesh,
        compiler_params=pltpu.CompilerParams(use_tc_tiling_on_sc=False),
        out_type=jax.ShapeDtypeStruct((nc, n_acc, w), jnp.float32),
        scratch_types=[
            pltpu.VMEM((chunk,), jnp.int32),
            pltpu.VMEM((chunk,), jnp.int32),
            pltpu.VMEM((chunk, w), jnp.float32),
            pltpu.VMEM_SHARED((n_acc, w), jnp.float32),
            pltpu.SemaphoreType.DMA,
        ],
    )
    def k(t_hbm, src_hbm, dst_hbm, out_hbm, sidx, didx, rows, acc_sh, sem):
        cid = lax.axis_index("c")
        sid = lax.axis_index("s")
        wid = sid * nc + cid

        def zero_row(i, carry):
            for j in range(w // 16):
                rows[i, pl.ds(16 * j, 16)] = jnp.zeros((16,), jnp.float32)
            return carry
        lax.fori_loop(0, chunk, zero_row, 0)
        for j in range(rows_per_tile // chunk):
            pltpu.sync_copy(
                rows, acc_sh.at[pl.ds(sid * rows_per_tile + j * chunk, chunk)])
        plsc.subcore_barrier()

        def body(i, carry):
            base = wid * per_w + i * chunk
            pltpu.sync_copy(src_hbm.at[pl.ds(base, chunk)], sidx)
            pltpu.async_copy(t_hbm.at[sidx], rows, sem).wait()
            pltpu.sync_copy(dst_hbm.at[pl.ds(base, chunk)], didx)
            pltpu.sync_copy(rows, acc_sh.at[didx], add=True)
            return carry
        lax.fori_loop(0, n_chunks, body, 0)
        plsc.subcore_barrier()

        pltpu.sync_copy(
            acc_sh.at[pl.ds(sid * rows_per_tile, rows_per_tile)],
            out_hbm.at[cid, pl.ds(sid * rows_per_tile, rows_per_tile)])

    return k


def _segsum_sc(t, src, dst, n_out):
    n_rows, w = t.shape
    e = src.shape[0]
    n_acc = 6144
    e_pad = -(-e // 4096) * 4096
    src = jnp.clip(src, 0, n_rows - 1)
    dst = jnp.where((dst >= 0) & (dst < n_out), dst, n_acc - 1)
    src = jnp.concatenate([src, jnp.zeros((e_pad - e,), jnp.int32)])
    dst = jnp.concatenate(
        [dst, jnp.full((e_pad - e,), n_acc - 1, jnp.int32)])
    parts = _make_segsum(n_rows, n_acc, e_pad, w)(t, src, dst)
    return (parts[0] + parts[1])[:n_out]


def _head_body(xd2_ref, mask_ref, dw_ref, db_ref, o_ref):
    xd1 = xd2_ref[...] * mask_ref[...]
    h = jax.nn.relu(
        jnp.dot(xd1, dw_ref[...], preferred_element_type=jnp.float32)
        + db_ref[...])
    n = xd2_ref.shape[0]
    o_ref[...] = jnp.sum(h, axis=0, keepdims=True) * (1.0 / n)


def _head(xd2, mask, dw, db):
    return pl.pallas_call(
        _head_body,
        out_shape=jax.ShapeDtypeStruct((1, dw.shape[1]), jnp.float32),
    )(xd2, mask[:, None], dw, db[None, :])


def _bn(h, g, b):
    m = jnp.mean(h, axis=0)
    v = jnp.var(h, axis=0)
    return (h - m) / jnp.sqrt(v + 1e-5) * g + b


def _gin(x, ei, p, n):
    src, dst = ei[0], ei[1]
    agg = jax.ops.segment_sum(x[src], dst, num_segments=n)
    h = x + agg
    h = jax.nn.relu(_bn(h @ p["W1"] + p["b1"], p["g1"], p["be1"]))
    h = jax.nn.relu(_bn(h @ p["W2"] + p["b2"], p["g2"], p["be2"]))
    return h


def _approx_eigvecs(ei, n, seed, iters=15):
    src, dst = ei[0], ei[1]
    s2 = jnp.concatenate([src, dst])
    d2 = jnp.concatenate([dst, src])
    deg = jax.ops.segment_sum(jnp.ones(s2.shape[0], jnp.float32), d2,
                              num_segments=n)
    dis = 1.0 / jnp.sqrt(jnp.maximum(deg, 1.0))

    def apply_l(q):
        msg = dis[s2][:, None] * q[s2]
        agg = jax.ops.segment_sum(msg, d2, num_segments=n)
        return q - dis[:, None] * agg

    q = jax.random.normal(jax.random.key(seed), (n, 3), dtype=jnp.float32)
    for _ in range(iters):
        q, _ = jnp.linalg.qr(apply_l(q))
    return q


def _spect_pool(ei, h, pp, ratio, seed):
    n = h.shape[0]
    la = jax.lax.stop_gradient(_approx_eigvecs(ei, n, seed))
    fw = h @ pp["Wf"] + pp["bf"]
    sw = la @ pp["Ws"] + pp["bs"]
    w = jnp.concatenate([fw, sw], axis=1) @ pp["Wp"] + pp["bp"]
    scores = jax.nn.sigmoid(w[:, 0])
    k = max(1, int(math.ceil(ratio * n)))
    vals, idx = jax.lax.top_k(scores, k)
    h_new = h[idx] * vals[:, None]
    ei_new = ei[:, idx]
    return h_new, idx, ei_new


def _gin_tail_sc(x, ei, p, n):
    src, dst = ei[0], ei[1]
    agg = _segsum_sc(x, src, dst, n)
    h = x + agg
    h = jax.nn.relu(_bn(h @ p["W1"] + p["b1"], p["g1"], p["be1"]))
    h = jax.nn.relu(_bn(h @ p["W2"] + p["b2"], p["g2"], p["be2"]))
    return h


def kernel(x, edge_index, batch, params):
    n = x.shape[0]
    x1 = jax.nn.relu(_gin(x, edge_index, params["conv1"], n))
    x1p, idx1, ei1 = _spect_pool(edge_index, x1, params["pool1"], 0.8, 1)
    x2 = jax.nn.relu(_gin(x1p, ei1, params["conv2"], x1p.shape[0]))
    x2p, idx2, ei2 = _spect_pool(ei1, x2, params["pool2"], 0.8, 2)

    xm = _gin_tail_sc(x2p, ei2, params["midconv"], x2p.shape[0])
    xd2 = jnp.zeros((n, xm.shape[1]), xm.dtype).at[idx2].set(xm)
    xd2 = _gin_tail_sc(xd2, ei2, params["decoder2"], n)
    mask = jnp.zeros((n,), jnp.float32).at[idx1].set(1.0)
    return _head(xd2, mask, params["dec1_W"], params["dec1_b"])
